# baseline (speedup 1.0000x reference)
import jax
import jax.numpy as jnp
from jax import lax
from jax.experimental import pallas as pl
from jax.experimental.pallas import tpu as pltpu
from jax.experimental.pallas import tpu_sc as plsc

N = 10000
E = 320000
H = 128
EMB = 128
NCLS = 5

NC = 2
NS = 16
NW = NC * NS

ROWS_PER_W = 320
NPAD = NW * ROWS_PER_W
G_CHUNK = 64
G_STEPS = ROWS_PER_W // G_CHUNK

E_CHUNK = 128
E_STEPS = 162
E_PER_TILE = E_STEPS * E_CHUNK
EPAD = NS * E_PER_TILE
NBUF = 3
DBUF = 6
JUNK_ROW = N
ACC_ROWS = N + 16
OUT_PER_TILE = 632
OUT_LAST = N - 15 * OUT_PER_TILE

TC_BLOCK = 2000
TC_GRID = N // TC_BLOCK


def _sc_mesh():
    return plsc.VectorSubcoreMesh(
        core_axis_name="c", subcore_axis_name="s", num_cores=NC, num_subcores=NS
    )


def _emb_gather(feats_pad, emb):
    def body(feats_hbm, emb_hbm, x_hbm, fidx, frows, sem):
        cid = lax.axis_index("c")
        sid = lax.axis_index("s")
        base = (sid * NC + cid) * ROWS_PER_W

        @pl.loop(0, G_STEPS)
        def _(j):
            off = base + j * G_CHUNK
            pltpu.sync_copy(feats_hbm.at[pl.ds(off, G_CHUNK)], fidx)
            pltpu.async_copy(emb_hbm.at[fidx], frows, sem).wait()
            pltpu.sync_copy(frows, x_hbm.at[pl.ds(off, G_CHUNK)])

    return pl.kernel(
        body,
        out_type=jax.ShapeDtypeStruct((NPAD, EMB), jnp.float32),
        mesh=_sc_mesh(),
        scratch_types=[
            pltpu.VMEM((G_CHUNK,), jnp.int32),
            pltpu.VMEM((G_CHUNK, EMB), jnp.float32),
            pltpu.SemaphoreType.DMA,
        ],
    )(feats_pad, emb)


def _node_round1(x, W_iou, b_iou, U_f_W, U_f_b2):
    def body(x_ref, wiou_ref, biou_ref, ufw_ref, ufb_ref, out_ref):
        iou = (
            jnp.dot(x_ref[...], wiou_ref[...], preferred_element_type=jnp.float32)
            + biou_ref[...]
        )
        i_g = jax.nn.sigmoid(iou[:, :H])
        o_g = jax.nn.sigmoid(iou[:, H : 2 * H])
        u_g = jnp.tanh(iou[:, 2 * H :])
        c1 = i_g * u_g
        h1 = o_g * jnp.tanh(c1)
        f1 = jax.nn.sigmoid(
            jnp.dot(h1, ufw_ref[...], preferred_element_type=jnp.float32)
            + ufb_ref[...]
        )
        out_ref[0] = h1
        out_ref[1] = f1 * c1

    return pl.pallas_call(
        body,
        grid=(TC_GRID,),
        in_specs=[
            pl.BlockSpec((TC_BLOCK, EMB), lambda i: (i, 0)),
            pl.BlockSpec((EMB, 3 * H), lambda i: (0, 0)),
            pl.BlockSpec((1, 3 * H), lambda i: (0, 0)),
            pl.BlockSpec((H, H), lambda i: (0, 0)),
            pl.BlockSpec((1, H), lambda i: (0, 0)),
        ],
        out_specs=pl.BlockSpec((2, TC_BLOCK, H), lambda i: (0, i, 0)),
        out_shape=jax.ShapeDtypeStruct((2, N, H), jnp.float32),
    )(x, W_iou, b_iou, U_f_W, U_f_b2)


def _edge_reduce(nodes_flat, srcs2, dst_pad):
    def body(nodes_hbm, src_hbm, dst_hbm, out_hbm, sidx, didx, rows, acc,
             gsem0, gsem1, gsem2, ssem0, ssem1, ssem2,
             isem0, isem1, isem2, isem3, isem4, isem5):
        cid = lax.axis_index("c")
        sid = lax.axis_index("s")
        gsem = (gsem0, gsem1, gsem2)
        ssem = (ssem0, ssem1, ssem2)
        isem = (isem0, isem1, isem2, isem3, isem4, isem5)
        s0 = cid * EPAD + sid * E_PER_TILE
        d0 = sid * E_PER_TILE

        @pl.loop(0, 128)
        def _(i):
            for k in range(H // 16):
                rows[0, i, pl.ds(k * 16, 16)] = jnp.zeros((16,), jnp.float32)

        zbase = sid * OUT_PER_TILE

        @pl.loop(0, 4)
        def _(k):
            pltpu.sync_copy(rows.at[0], acc.at[pl.ds(zbase + k * 128, 128)])

        @pl.when(sid < NS - 1)
        def _():
            pltpu.sync_copy(
                rows.at[0, pl.ds(0, OUT_PER_TILE - 512)],
                acc.at[pl.ds(zbase + 512, OUT_PER_TILE - 512)],
            )

        @pl.when(sid == NS - 1)
        def _():
            pltpu.sync_copy(
                rows.at[0, pl.ds(0, OUT_LAST - 512 + ACC_ROWS - N)],
                acc.at[pl.ds(zbase + 512, OUT_LAST - 512 + ACC_ROWS - N)],
            )

        plsc.subcore_barrier()

        def i_start(j, k):
            pltpu.make_async_copy(
                src_hbm.at[pl.ds(s0 + j * E_CHUNK, E_CHUNK)],
                sidx.at[k % NBUF], isem[k % DBUF],
            ).start()
            pltpu.make_async_copy(
                dst_hbm.at[pl.ds(d0 + j * E_CHUNK, E_CHUNK)],
                didx.at[k % DBUF], isem[k % DBUF],
            ).start()

        def i_wait(k):
            pltpu.make_async_copy(
                src_hbm.at[pl.ds(s0, E_CHUNK)], sidx.at[k % NBUF],
                isem[k % DBUF],
            ).wait()
            pltpu.make_async_copy(
                dst_hbm.at[pl.ds(d0, E_CHUNK)], didx.at[k % DBUF],
                isem[k % DBUF],
            ).wait()

        def g_desc(k):
            b = k % NBUF
            return pltpu.make_async_copy(
                nodes_hbm.at[sidx.at[b]], rows.at[b], gsem[b]
            )

        def s_desc(k):
            return pltpu.make_async_copy(
                rows.at[k % NBUF], acc.at[didx.at[k % DBUF]], ssem[k % NBUF]
            )

        def chunk_body(j, k, s_wait_prev, start_next_g, start_next_i):
            if start_next_g:
                i_wait(k + 2)
                if s_wait_prev:
                    s_desc(k + 2).wait()
                g_desc(k + 2).start()
            g_desc(k).wait()
            s_desc(k).start(add=True)
            if start_next_i:
                i_start(j + NBUF, k + NBUF)

        for k in range(NBUF):
            i_start(k, k)
        i_wait(0)
        g_desc(0).start()
        i_wait(1)
        g_desc(1).start()

        for k in range(DBUF):
            chunk_body(k, k, k > 0, True, True)

        @pl.loop(1, E_STEPS // DBUF - 1)
        def _(p):
            for k in range(DBUF):
                chunk_body(p * DBUF + k, k, True, True, True)

        j0 = E_STEPS - DBUF
        for k in range(DBUF):
            chunk_body(j0 + k, k, True, k < DBUF - 2, k < NBUF)
        for k in range(NBUF):
            s_desc(k).wait()

        plsc.subcore_barrier()

        obase = cid * N + sid * OUT_PER_TILE

        @pl.when(sid < NS - 1)
        def _():
            pltpu.sync_copy(
                acc.at[pl.ds(zbase, OUT_PER_TILE)],
                out_hbm.at[pl.ds(obase, OUT_PER_TILE)],
            )

        @pl.when(sid == NS - 1)
        def _():
            pltpu.sync_copy(
                acc.at[pl.ds(zbase, OUT_LAST)],
                out_hbm.at[pl.ds(obase, OUT_LAST)],
            )

    return pl.kernel(
        body,
        out_type=jax.ShapeDtypeStruct((2 * N, H), jnp.float32),
        mesh=_sc_mesh(),
        scratch_types=[
            pltpu.VMEM((NBUF, E_CHUNK), jnp.int32),
            pltpu.VMEM((DBUF, E_CHUNK), jnp.int32),
            pltpu.VMEM((NBUF, E_CHUNK, H), jnp.float32),
            pltpu.VMEM_SHARED((ACC_ROWS, H), jnp.float32),
        ] + [pltpu.SemaphoreType.DMA] * 12,
    )(nodes_flat, srcs2, dst_pad)


def _node_round2(x, h_sum, c_agg, W_iou, U_iou, b_iou, lin_W, lin_b2):
    def body(x_ref, hs_ref, ca_ref, wiou_ref, uiou_ref, biou_ref, lw_ref,
             lb_ref, out_ref, acc_ref):
        i = pl.program_id(0)
        iou = (
            jnp.dot(x_ref[...], wiou_ref[...], preferred_element_type=jnp.float32)
            + jnp.dot(hs_ref[...], uiou_ref[...], preferred_element_type=jnp.float32)
            + biou_ref[...]
        )
        i_g = jax.nn.sigmoid(iou[:, :H])
        o_g = jax.nn.sigmoid(iou[:, H : 2 * H])
        u_g = jnp.tanh(iou[:, 2 * H :])
        c2 = i_g * u_g + ca_ref[...]
        h2 = o_g * jnp.tanh(c2)
        part = jnp.sum(h2, axis=0, keepdims=True)

        @pl.when(i == 0)
        def _():
            acc_ref[...] = jnp.zeros_like(acc_ref)

        acc_ref[...] += part

        @pl.when(i == TC_GRID - 1)
        def _():
            hg = acc_ref[...] * (1.0 / N)
            out_ref[...] = (
                jnp.dot(hg, lw_ref[...], preferred_element_type=jnp.float32)
                + lb_ref[...]
            )

    return pl.pallas_call(
        body,
        grid=(TC_GRID,),
        in_specs=[
            pl.BlockSpec((TC_BLOCK, EMB), lambda i: (i, 0)),
            pl.BlockSpec((TC_BLOCK, H), lambda i: (i, 0)),
            pl.BlockSpec((TC_BLOCK, H), lambda i: (i, 0)),
            pl.BlockSpec((EMB, 3 * H), lambda i: (0, 0)),
            pl.BlockSpec((H, 3 * H), lambda i: (0, 0)),
            pl.BlockSpec((1, 3 * H), lambda i: (0, 0)),
            pl.BlockSpec((H, NCLS), lambda i: (0, 0)),
            pl.BlockSpec((1, NCLS), lambda i: (0, 0)),
        ],
        out_specs=pl.BlockSpec((1, NCLS), lambda i: (0, 0)),
        out_shape=jax.ShapeDtypeStruct((1, NCLS), jnp.float32),
        scratch_shapes=[pltpu.VMEM((1, H), jnp.float32)],
    )(x, h_sum, c_agg, W_iou, U_iou, b_iou, lin_W, lin_b2)


def kernel(features, edge_index, emb, W_iou, U_iou, b_iou, U_f_W, U_f_b,
           lin_W, lin_b):
    feats_pad = jnp.concatenate(
        [features.astype(jnp.int32), jnp.zeros((NPAD - N,), jnp.int32)]
    )
    src = edge_index[0]
    dst = edge_index[1]
    src_pad = jnp.concatenate([src, jnp.zeros((EPAD - E,), jnp.int32)])
    dst_pad = jnp.concatenate(
        [dst, jnp.full((EPAD - E,), JUNK_ROW, jnp.int32)]
    )
    srcs2 = jnp.concatenate([src_pad, src_pad + N])

    x = _emb_gather(feats_pad, emb)
    nodes = _node_round1(x, W_iou, b_iou, U_f_W, U_f_b.reshape(1, H))
    nodes_flat = nodes.reshape(2 * N, H)
    red = _edge_reduce(nodes_flat, srcs2, dst_pad)
    h_sum = red[:N]
    c_agg = red[N:]
    logits = _node_round2(
        x, h_sum, c_agg, W_iou, U_iou, b_iou, lin_W, lin_b.reshape(1, NCLS)
    )
    return logits.reshape(NCLS)


# speedup vs baseline: 1.4533x; 1.4533x over previous
import jax
import jax.numpy as jnp
from jax import lax
from jax.experimental import pallas as pl
from jax.experimental.pallas import tpu as pltpu
from jax.experimental.pallas import tpu_sc as plsc

N = 10000
E = 320000
H = 128
EMB = 128
NCLS = 5

NC = 2
NS = 16
NW = NC * NS

ROWS_PER_W = 320
NPAD = NW * ROWS_PER_W
G_CHUNK = 64
G_STEPS = ROWS_PER_W // G_CHUNK

E_CHUNK = 128
E_STEPS = 159
E_PER_TILE = E_STEPS * E_CHUNK
EPAD = NS * E_PER_TILE
NBUF = 3
JUNK_ROW = N
ACC_ROWS = N + 16
OUT_PER_TILE = 632
OUT_LAST = N - 15 * OUT_PER_TILE

TC_BLOCK = 2000
TC_GRID = N // TC_BLOCK


def _sc_mesh():
    return plsc.VectorSubcoreMesh(
        core_axis_name="c", subcore_axis_name="s", num_cores=NC, num_subcores=NS
    )


def _emb_gather(feats_pad, emb):
    def body(feats_hbm, emb_hbm, x_hbm, fidx, frows, sem):
        cid = lax.axis_index("c")
        sid = lax.axis_index("s")
        base = (sid * NC + cid) * ROWS_PER_W

        @pl.loop(0, G_STEPS)
        def _(j):
            off = base + j * G_CHUNK
            pltpu.sync_copy(feats_hbm.at[pl.ds(off, G_CHUNK)], fidx)
            pltpu.async_copy(emb_hbm.at[fidx], frows, sem).wait()
            pltpu.sync_copy(frows, x_hbm.at[pl.ds(off, G_CHUNK)])

    return pl.kernel(
        body,
        out_type=jax.ShapeDtypeStruct((NPAD, EMB), jnp.float32),
        mesh=_sc_mesh(),
        scratch_types=[
            pltpu.VMEM((G_CHUNK,), jnp.int32),
            pltpu.VMEM((G_CHUNK, EMB), jnp.float32),
            pltpu.SemaphoreType.DMA,
        ],
    )(feats_pad, emb)


def _node_round1(x, W_iou, b_iou, U_f_W, U_f_b2):
    def body(x_ref, wiou_ref, biou_ref, ufw_ref, ufb_ref, out_ref):
        iou = (
            jnp.dot(x_ref[...], wiou_ref[...], preferred_element_type=jnp.float32)
            + biou_ref[...]
        )
        i_g = jax.nn.sigmoid(iou[:, :H])
        o_g = jax.nn.sigmoid(iou[:, H : 2 * H])
        u_g = jnp.tanh(iou[:, 2 * H :])
        c1 = i_g * u_g
        h1 = o_g * jnp.tanh(c1)
        f1 = jax.nn.sigmoid(
            jnp.dot(h1, ufw_ref[...], preferred_element_type=jnp.float32)
            + ufb_ref[...]
        )
        out_ref[0] = h1
        out_ref[1] = f1 * c1

    return pl.pallas_call(
        body,
        grid=(TC_GRID,),
        in_specs=[
            pl.BlockSpec((TC_BLOCK, EMB), lambda i: (i, 0)),
            pl.BlockSpec((EMB, 3 * H), lambda i: (0, 0)),
            pl.BlockSpec((1, 3 * H), lambda i: (0, 0)),
            pl.BlockSpec((H, H), lambda i: (0, 0)),
            pl.BlockSpec((1, H), lambda i: (0, 0)),
        ],
        out_specs=pl.BlockSpec((2, TC_BLOCK, H), lambda i: (0, i, 0)),
        out_shape=jax.ShapeDtypeStruct((2, N, H), jnp.float32),
    )(x, W_iou, b_iou, U_f_W, U_f_b2)


def _edge_reduce(nodes_flat, srcs2, dst_pad):
    def body(nodes_hbm, src_hbm, dst_hbm, out_hbm, sidx, didx, rows, acc,
             gsem0, gsem1, gsem2, ssem, isem0, isem1, isem2):
        cid = lax.axis_index("c")
        sid = lax.axis_index("s")
        gsem = (gsem0, gsem1, gsem2)
        isem = (isem0, isem1, isem2)
        s0 = cid * EPAD + sid * E_PER_TILE
        d0 = sid * E_PER_TILE

        @pl.loop(0, 128)
        def _(i):
            for k in range(H // 16):
                rows[0, i, pl.ds(k * 16, 16)] = jnp.zeros((16,), jnp.float32)

        zbase = sid * OUT_PER_TILE

        @pl.loop(0, 4)
        def _(k):
            pltpu.sync_copy(rows.at[0], acc.at[pl.ds(zbase + k * 128, 128)])

        @pl.when(sid < NS - 1)
        def _():
            pltpu.sync_copy(
                rows.at[0, pl.ds(0, OUT_PER_TILE - 512)],
                acc.at[pl.ds(zbase + 512, OUT_PER_TILE - 512)],
            )

        @pl.when(sid == NS - 1)
        def _():
            pltpu.sync_copy(
                rows.at[0, pl.ds(0, OUT_LAST - 512 + ACC_ROWS - N)],
                acc.at[pl.ds(zbase + 512, OUT_LAST - 512 + ACC_ROWS - N)],
            )

        plsc.subcore_barrier()

        def i_start(j, b):
            pltpu.make_async_copy(
                src_hbm.at[pl.ds(s0 + j * E_CHUNK, E_CHUNK)], sidx.at[b], isem[b]
            ).start()
            pltpu.make_async_copy(
                dst_hbm.at[pl.ds(d0 + j * E_CHUNK, E_CHUNK)], didx.at[b], isem[b]
            ).start()

        def i_wait(b):
            pltpu.make_async_copy(
                src_hbm.at[pl.ds(s0, E_CHUNK)], sidx.at[b], isem[b]
            ).wait()
            pltpu.make_async_copy(
                dst_hbm.at[pl.ds(d0, E_CHUNK)], didx.at[b], isem[b]
            ).wait()

        def g_desc(b):
            return pltpu.make_async_copy(
                nodes_hbm.at[sidx.at[b]], rows.at[b], gsem[b]
            )

        def s_desc(b):
            return pltpu.make_async_copy(rows.at[b], acc.at[didx.at[b]], ssem)

        for b in range(NBUF):
            i_start(b, b)
        i_wait(0)
        g_desc(0).start()
        i_wait(1)
        g_desc(1).start()

        @pl.loop(0, E_STEPS // NBUF - 1)
        def _(p):
            for b in range(NBUF):
                j = p * NBUF + b
                b2 = (b + 2) % NBUF
                i_wait(b2)
                g_desc(b2).start()
                g_desc(b).wait()
                s_desc(b).start(add=True)
                s_desc(b).wait()
                i_start(j + NBUF, b)

        i_wait(2)
        g_desc(2).start()
        g_desc(0).wait()
        s_desc(0).start(add=True)
        s_desc(0).wait()
        g_desc(1).wait()
        s_desc(1).start(add=True)
        s_desc(1).wait()
        g_desc(2).wait()
        s_desc(2).start(add=True)
        s_desc(2).wait()

        plsc.subcore_barrier()

        obase = cid * N + sid * OUT_PER_TILE

        @pl.when(sid < NS - 1)
        def _():
            pltpu.sync_copy(
                acc.at[pl.ds(zbase, OUT_PER_TILE)],
                out_hbm.at[pl.ds(obase, OUT_PER_TILE)],
            )

        @pl.when(sid == NS - 1)
        def _():
            pltpu.sync_copy(
                acc.at[pl.ds(zbase, OUT_LAST)],
                out_hbm.at[pl.ds(obase, OUT_LAST)],
            )

    return pl.kernel(
        body,
        out_type=jax.ShapeDtypeStruct((2 * N, H), jnp.float32),
        mesh=_sc_mesh(),
        scratch_types=[
            pltpu.VMEM((NBUF, E_CHUNK), jnp.int32),
            pltpu.VMEM((NBUF, E_CHUNK), jnp.int32),
            pltpu.VMEM((NBUF, E_CHUNK, H), jnp.float32),
            pltpu.VMEM_SHARED((ACC_ROWS, H), jnp.float32),
            pltpu.SemaphoreType.DMA,
            pltpu.SemaphoreType.DMA,
            pltpu.SemaphoreType.DMA,
            pltpu.SemaphoreType.DMA,
            pltpu.SemaphoreType.DMA,
            pltpu.SemaphoreType.DMA,
            pltpu.SemaphoreType.DMA,
        ],
    )(nodes_flat, srcs2, dst_pad)


def _node_round2(x, h_sum, c_agg, W_iou, U_iou, b_iou, lin_W, lin_b2):
    def body(x_ref, hs_ref, ca_ref, wiou_ref, uiou_ref, biou_ref, lw_ref,
             lb_ref, out_ref, acc_ref):
        i = pl.program_id(0)
        iou = (
            jnp.dot(x_ref[...], wiou_ref[...], preferred_element_type=jnp.float32)
            + jnp.dot(hs_ref[...], uiou_ref[...], preferred_element_type=jnp.float32)
            + biou_ref[...]
        )
        i_g = jax.nn.sigmoid(iou[:, :H])
        o_g = jax.nn.sigmoid(iou[:, H : 2 * H])
        u_g = jnp.tanh(iou[:, 2 * H :])
        c2 = i_g * u_g + ca_ref[...]
        h2 = o_g * jnp.tanh(c2)
        part = jnp.sum(h2, axis=0, keepdims=True)

        @pl.when(i == 0)
        def _():
            acc_ref[...] = jnp.zeros_like(acc_ref)

        acc_ref[...] += part

        @pl.when(i == TC_GRID - 1)
        def _():
            hg = acc_ref[...] * (1.0 / N)
            out_ref[...] = (
                jnp.dot(hg, lw_ref[...], preferred_element_type=jnp.float32)
                + lb_ref[...]
            )

    return pl.pallas_call(
        body,
        grid=(TC_GRID,),
        in_specs=[
            pl.BlockSpec((TC_BLOCK, EMB), lambda i: (i, 0)),
            pl.BlockSpec((TC_BLOCK, H), lambda i: (i, 0)),
            pl.BlockSpec((TC_BLOCK, H), lambda i: (i, 0)),
            pl.BlockSpec((EMB, 3 * H), lambda i: (0, 0)),
            pl.BlockSpec((H, 3 * H), lambda i: (0, 0)),
            pl.BlockSpec((1, 3 * H), lambda i: (0, 0)),
            pl.BlockSpec((H, NCLS), lambda i: (0, 0)),
            pl.BlockSpec((1, NCLS), lambda i: (0, 0)),
        ],
        out_specs=pl.BlockSpec((1, NCLS), lambda i: (0, 0)),
        out_shape=jax.ShapeDtypeStruct((1, NCLS), jnp.float32),
        scratch_shapes=[pltpu.VMEM((1, H), jnp.float32)],
    )(x, h_sum, c_agg, W_iou, U_iou, b_iou, lin_W, lin_b2)


def kernel(features, edge_index, emb, W_iou, U_iou, b_iou, U_f_W, U_f_b,
           lin_W, lin_b):
    feats_pad = jnp.concatenate(
        [features.astype(jnp.int32), jnp.zeros((NPAD - N,), jnp.int32)]
    )
    src = edge_index[0]
    dst = edge_index[1]
    src_pad = jnp.concatenate([src, jnp.zeros((EPAD - E,), jnp.int32)])
    dst_pad = jnp.concatenate(
        [dst, jnp.full((EPAD - E,), JUNK_ROW, jnp.int32)]
    )
    srcs2 = jnp.concatenate([src_pad, src_pad + N])

    x = _emb_gather(feats_pad, emb)
    nodes = _node_round1(x, W_iou, b_iou, U_f_W, U_f_b.reshape(1, H))
    nodes_flat = nodes.reshape(2 * N, H)
    red = _edge_reduce(nodes_flat, srcs2, dst_pad)
    h_sum = red[:N]
    c_agg = red[N:]
    logits = _node_round2(
        x, h_sum, c_agg, W_iou, U_iou, b_iou, lin_W, lin_b.reshape(1, NCLS)
    )
    return logits.reshape(NCLS)
